# baseline (speedup 1.0000x reference)
import functools

import jax
import jax.numpy as jnp
from jax import lax
from jax.experimental import pallas as pl
from jax.experimental.pallas import tpu as pltpu
from jax.experimental.pallas import tpu_sc as plsc

D_MODEL = 1024
NUM = 16384

NUM_CORES = 2
NUM_SUBCORES = 16
NUM_WORKERS = NUM_CORES * NUM_SUBCORES

B_PER_W = NUM // NUM_WORKERS
CHUNK = 16
NCHUNKS = B_PER_W // CHUNK


def _gather_body(table_hbm, idx_hbm, out_hbm, idx_v, rows_v, sp_rows,
                 semg0, semg1, semw0, semw1):
    sid = lax.axis_index("s")
    wid = sid * NUM_CORES + lax.axis_index("c")
    base = wid * B_PER_W
    semg = (semg0, semg1)
    semw = (semw0, semw1)

    pltpu.sync_copy(idx_hbm.at[pl.ds(wid * NCHUNKS, NCHUNKS)], idx_v)

    def start_gather(g, b):
        pltpu.async_copy(table_hbm.at[idx_v.at[g]], rows_v.at[b], semg[b])

    def wait_gather(b):
        pltpu.make_async_copy(
            table_hbm.at[pl.ds(0, CHUNK)], rows_v.at[b], semg[b]
        ).wait()

    def start_writeback(g, p):
        pltpu.async_copy(
            sp_rows.at[sid, p], out_hbm.at[pl.ds(base + g * CHUNK, CHUNK)],
            semw[p],
        )

    def wait_writeback(p):
        pltpu.make_async_copy(
            sp_rows.at[sid, p], out_hbm.at[pl.ds(base, CHUNK)], semw[p]
        ).wait()

    start_gather(0, 0)
    for i in range(NCHUNKS):
        b = i % 2
        wait_gather(b)
        if i + 1 < NCHUNKS:
            start_gather(i + 1, 1 - b)
        if i >= 2:
            wait_writeback(b)
        pltpu.sync_copy(rows_v.at[b], sp_rows.at[sid, b])
        start_writeback(i, b)
    wait_writeback(0)
    wait_writeback(1)


@jax.jit
def _positional_gather(encoding, idx):
    kernel_fn = pl.kernel(
        _gather_body,
        out_type=jax.ShapeDtypeStruct((NUM, D_MODEL), jnp.float32),
        mesh=plsc.VectorSubcoreMesh(core_axis_name="c", subcore_axis_name="s"),
        scratch_types=[
            pltpu.VMEM((NCHUNKS, CHUNK), jnp.int32),
            pltpu.VMEM((2, CHUNK, D_MODEL), jnp.float32),
            pltpu.VMEM_SHARED((NUM_SUBCORES, 2, CHUNK, D_MODEL), jnp.float32),
            pltpu.SemaphoreType.DMA,
            pltpu.SemaphoreType.DMA,
            pltpu.SemaphoreType.DMA,
            pltpu.SemaphoreType.DMA,
        ],
    )
    return kernel_fn(encoding, idx)


def kernel(encoding, t):
    idx = t.reshape(NUM).astype(jnp.int32).reshape(NUM // CHUNK, CHUNK)
    return _positional_gather(encoding, idx)


# speedup vs baseline: 1.0845x; 1.0845x over previous
import jax
import jax.numpy as jnp
from jax import lax
from jax.experimental import pallas as pl
from jax.experimental.pallas import tpu as pltpu
from jax.experimental.pallas import tpu_sc as plsc

D_MODEL = 1024
NUM = 16384

NUM_CORES = 2
NUM_SUBCORES = 16
NUM_WORKERS = NUM_CORES * NUM_SUBCORES

B_PER_W = NUM // NUM_WORKERS
CHUNK = 32
NCHUNKS = B_PER_W // CHUNK


def _gather_body(table_hbm, idx_hbm, out_hbm, idx_v, rows_v, sem0, sem1):
    wid = lax.axis_index("s") * NUM_CORES + lax.axis_index("c")
    base = wid * B_PER_W
    sems = (sem0, sem1)

    pltpu.sync_copy(idx_hbm.at[pl.ds(wid * NCHUNKS, NCHUNKS)], idx_v)

    def start(g, b):
        pltpu.async_copy(table_hbm.at[idx_v.at[g]], rows_v.at[b], sems[b])

    def finish(g, b):
        pltpu.make_async_copy(
            table_hbm.at[pl.ds(0, CHUNK)], rows_v.at[b], sems[b]
        ).wait()
        pltpu.sync_copy(rows_v.at[b], out_hbm.at[pl.ds(base + g * CHUNK, CHUNK)])

    start(0, 0)
    for g in range(NCHUNKS):
        b = g % 2
        if g + 1 < NCHUNKS:
            start(g + 1, 1 - b)
        finish(g, b)


@jax.jit
def _positional_gather(encoding, idx):
    kernel_fn = pl.kernel(
        _gather_body,
        out_type=jax.ShapeDtypeStruct((NUM, D_MODEL), jnp.float32),
        mesh=plsc.VectorSubcoreMesh(core_axis_name="c", subcore_axis_name="s"),
        scratch_types=[
            pltpu.VMEM((NCHUNKS, CHUNK), jnp.int32),
            pltpu.VMEM((2, CHUNK, D_MODEL), jnp.float32),
            pltpu.SemaphoreType.DMA,
            pltpu.SemaphoreType.DMA,
        ],
    )
    return kernel_fn(encoding, idx)


def kernel(encoding, t):
    idx = t.reshape(NUM // CHUNK, CHUNK).astype(jnp.int32)
    return _positional_gather(encoding, idx)


# speedup vs baseline: 1.1152x; 1.0283x over previous
import jax
import jax.numpy as jnp
from jax import lax
from jax.experimental import pallas as pl
from jax.experimental.pallas import tpu as pltpu
from jax.experimental.pallas import tpu_sc as plsc

D_MODEL = 1024
NUM = 16384

NUM_CORES = 2
NUM_SUBCORES = 16
NUM_WORKERS = NUM_CORES * NUM_SUBCORES

B_PER_W = NUM // NUM_WORKERS
CHUNK = 32
NCHUNKS = B_PER_W // CHUNK


def _gather_body(table_hbm, idx_hbm, out_hbm, idx_v, rows_v, sem0, sem1):
    wid = lax.axis_index("s") * NUM_CORES + lax.axis_index("c")
    base = wid * B_PER_W
    sems = (sem0, sem1)

    pltpu.sync_copy(idx_hbm.at[pl.ds(wid * NCHUNKS, NCHUNKS)], idx_v)

    def start(g, b):
        pltpu.async_copy(table_hbm.at[idx_v.at[g]], rows_v.at[b], sems[b])

    def finish(g, b):
        pltpu.make_async_copy(
            table_hbm.at[pl.ds(0, CHUNK)], rows_v.at[b], sems[b]
        ).wait()
        pltpu.sync_copy(rows_v.at[b], out_hbm.at[pl.ds(base + g * CHUNK, CHUNK)])

    start(0, 0)

    def body(i, carry):
        g0 = i * 2
        for b in range(2):
            g = g0 + b

            @pl.when(g + 1 < NCHUNKS)
            def _():
                start(g + 1, 1 - b)

            finish(g, b)
        return carry

    lax.fori_loop(0, NCHUNKS // 2, body, 0)


@jax.jit
def _positional_gather(encoding, idx):
    kernel_fn = pl.kernel(
        _gather_body,
        out_type=jax.ShapeDtypeStruct((NUM, D_MODEL), jnp.float32),
        mesh=plsc.VectorSubcoreMesh(core_axis_name="c", subcore_axis_name="s"),
        scratch_types=[
            pltpu.VMEM((NCHUNKS, CHUNK), jnp.int32),
            pltpu.VMEM((2, CHUNK, D_MODEL), jnp.float32),
            pltpu.SemaphoreType.DMA,
            pltpu.SemaphoreType.DMA,
        ],
    )
    return kernel_fn(encoding, idx)


def kernel(encoding, t):
    idx = t.reshape(NUM // CHUNK, CHUNK).astype(jnp.int32)
    return _positional_gather(encoding, idx)


# speedup vs baseline: 1.1166x; 1.0013x over previous
import jax
import jax.numpy as jnp
from jax import lax
from jax.experimental import pallas as pl
from jax.experimental.pallas import tpu as pltpu
from jax.experimental.pallas import tpu_sc as plsc

D_MODEL = 1024
NUM = 16384

NUM_CORES = 2
NUM_SUBCORES = 16
NUM_WORKERS = NUM_CORES * NUM_SUBCORES

B_PER_W = NUM // NUM_WORKERS
CHUNK = 32
NCHUNKS = B_PER_W // CHUNK


def _gather_body(table_hbm, idx_hbm, out_hbm, idx_v, rows_v, sem0, sem1):
    wid = lax.axis_index("s") * NUM_CORES + lax.axis_index("c")
    base = wid * B_PER_W
    sems = (sem0, sem1)

    pltpu.sync_copy(idx_hbm.at[pl.ds(base, B_PER_W)], idx_v)

    def start(g, b):
        pltpu.async_copy(
            table_hbm.at[idx_v.at[pl.ds(g * CHUNK, CHUNK)]], rows_v.at[b],
            sems[b],
        )

    def finish(g, b):
        pltpu.make_async_copy(
            table_hbm.at[pl.ds(0, CHUNK)], rows_v.at[b], sems[b]
        ).wait()
        pltpu.sync_copy(rows_v.at[b], out_hbm.at[pl.ds(base + g * CHUNK, CHUNK)])

    start(0, 0)

    def body(i, carry):
        g0 = i * 2
        for b in range(2):
            g = g0 + b

            @pl.when(g + 1 < NCHUNKS)
            def _():
                start(g + 1, 1 - b)

            finish(g, b)
        return carry

    lax.fori_loop(0, NCHUNKS // 2, body, 0)


@jax.jit
def _positional_gather(encoding, idx):
    kernel_fn = pl.kernel(
        _gather_body,
        out_type=jax.ShapeDtypeStruct((NUM, D_MODEL), jnp.float32),
        mesh=plsc.VectorSubcoreMesh(core_axis_name="c", subcore_axis_name="s"),
        scratch_types=[
            pltpu.VMEM((B_PER_W,), jnp.int32),
            pltpu.VMEM((2, CHUNK, D_MODEL), jnp.float32),
            pltpu.SemaphoreType.DMA,
            pltpu.SemaphoreType.DMA,
        ],
    )
    return kernel_fn(encoding, idx)


def kernel(encoding, t):
    idx = t.reshape(NUM).astype(jnp.int32)
    return _positional_gather(encoding, idx)
